# Initial kernel scaffold; baseline (speedup 1.0000x reference)
import jax
import jax.numpy as jnp
from jax.experimental import pallas as pl


def kernel(x, gate_w, gate_b, fc1_w, fc1_b, fc2_w, fc2_b, fcp_w, fcp_b, fca_w, fca_b):
    raise NotImplementedError("write your pallas kernel here")


# speedup vs baseline: 1.2126x; 1.2126x over previous
import functools

import jax
import jax.numpy as jnp
from jax.experimental import pallas as pl
from jax.experimental.pallas import tpu as pltpu

N, D, H, O, E = 4096, 1024, 1024, 128, 8
LANES = 128
A0_INIT, A0_MIN, A0_MAX = 10.0, 1.0, 500.0
TIL = 512
NT = N // TIL
NEG = -1e30


def _gating_body(x_ref, gw_ref, gb_ref, gates_ref, aux_ref, imp_acc, load_acc):
    i = pl.program_id(0)
    logits = (
        jnp.dot(x_ref[...], gw_ref[...], preferred_element_type=jnp.float32)
        + gb_ref[...]
    )
    lane = jax.lax.broadcasted_iota(jnp.int32, (TIL, LANES), 1)
    v0 = jnp.max(logits, axis=1, keepdims=True)
    i0 = jnp.min(jnp.where(logits == v0, lane, LANES), axis=1, keepdims=True)
    masked = jnp.where(lane == i0, NEG, logits)
    v1 = jnp.max(masked, axis=1, keepdims=True)
    i1 = jnp.min(jnp.where(masked == v1, lane, LANES), axis=1, keepdims=True)
    g0 = 1.0 / (1.0 + jnp.exp(v1 - v0))
    g1 = 1.0 / (1.0 + jnp.exp(v0 - v1))
    gates = jnp.where(lane == i0, g0, 0.0) + jnp.where(lane == i1, g1, 0.0)
    gates_ref[...] = gates

    @pl.when(i == 0)
    def _init():
        imp_acc[...] = jnp.zeros_like(imp_acc)
        load_acc[...] = jnp.zeros_like(load_acc)

    imp_acc[...] += jnp.sum(gates, axis=0, keepdims=True)
    load_acc[...] += jnp.sum((gates > 0.0).astype(jnp.float32), axis=0, keepdims=True)

    @pl.when(i == NT - 1)
    def _finish():
        lrow = jax.lax.broadcasted_iota(jnp.int32, (1, LANES), 1)
        m = lrow < E

        def std1_over_mean(v):
            mean = jnp.sum(jnp.where(m, v, 0.0)) / E
            var = jnp.sum(jnp.where(m, (v - mean) ** 2, 0.0)) / (E - 1)
            return jnp.sqrt(var) / (mean + 1e-8)

        aux = std1_over_mean(imp_acc[...]) + std1_over_mean(load_acc[...])
        aux_ref[...] = jnp.full((1, LANES), aux, jnp.float32)


def _moe_body(x_ref, gates_ref, w1_ref, b1_ref, w2_ref, b2_ref, wpc_ref, bpc_ref,
              out_ref, alpha_ref):
    e = pl.program_id(1)
    x = x_ref[...]
    h = jnp.maximum(
        jnp.dot(x, w1_ref[0], preferred_element_type=jnp.float32) + b1_ref[0], 0.0)
    h = jnp.maximum(
        jnp.dot(h, w2_ref[0], preferred_element_type=jnp.float32) + b2_ref[0], 0.0)
    z = jnp.dot(h, wpc_ref[0], preferred_element_type=jnp.float32) + bpc_ref[0]
    lane2 = jax.lax.broadcasted_iota(jnp.int32, (TIL, 2 * LANES), 1)
    zp = z[:, :O]
    za = jnp.sum(jnp.where(lane2 == O, z, 0.0), axis=1, keepdims=True)
    ex = jnp.exp(zp - jnp.max(zp, axis=1, keepdims=True))
    p = ex / jnp.sum(ex, axis=1, keepdims=True)
    sp = jnp.maximum(za, 0.0) + jnp.log(1.0 + jnp.exp(-jnp.abs(za)))
    a = jnp.clip(sp + A0_INIT, A0_MIN, A0_MAX)
    lane = jax.lax.broadcasted_iota(jnp.int32, (TIL, LANES), 1)
    ge = jnp.sum(jnp.where(lane == e, gates_ref[...], 0.0), axis=1, keepdims=True)

    @pl.when(e == 0)
    def _init():
        out_ref[...] = jnp.zeros_like(out_ref)
        alpha_ref[...] = jnp.zeros_like(alpha_ref)

    out_ref[...] += ge * p
    alpha_ref[...] += ge * a

    @pl.when(e == E - 1)
    def _norm():
        acc = out_ref[...]
        out_ref[...] = acc / (jnp.sum(acc, axis=1, keepdims=True) + 1e-8)


@jax.jit
def _run(x, gate_w, gate_b, fc1_w, fc1_b, fc2_w, fc2_b, fcp_w, fcp_b, fca_w, fca_b):
    gwp = jnp.zeros((D, LANES), jnp.float32).at[:, :E].set(gate_w)
    gbp = jnp.full((1, LANES), NEG, jnp.float32).at[0, :E].set(gate_b)

    gates, aux_vec = pl.pallas_call(
        _gating_body,
        grid=(NT,),
        in_specs=[
            pl.BlockSpec((TIL, D), lambda i: (i, 0)),
            pl.BlockSpec((D, LANES), lambda i: (0, 0)),
            pl.BlockSpec((1, LANES), lambda i: (0, 0)),
        ],
        out_specs=[
            pl.BlockSpec((TIL, LANES), lambda i: (i, 0)),
            pl.BlockSpec((1, LANES), lambda i: (0, 0)),
        ],
        out_shape=[
            jax.ShapeDtypeStruct((N, LANES), jnp.float32),
            jax.ShapeDtypeStruct((1, LANES), jnp.float32),
        ],
        scratch_shapes=[
            pltpu.VMEM((1, LANES), jnp.float32),
            pltpu.VMEM((1, LANES), jnp.float32),
        ],
    )(x, gwp, gbp)

    wpc = jnp.zeros((E, H, 2 * LANES), jnp.float32)
    wpc = wpc.at[:, :, :O].set(fcp_w).at[:, :, O:O + 1].set(fca_w)
    bpc = jnp.zeros((E, 1, 2 * LANES), jnp.float32)
    bpc = bpc.at[:, 0, :O].set(fcp_b).at[:, 0, O].set(fca_b[:, 0])

    p_hat, alpha = pl.pallas_call(
        _moe_body,
        grid=(NT, E),
        in_specs=[
            pl.BlockSpec((TIL, D), lambda i, e: (i, 0)),
            pl.BlockSpec((TIL, LANES), lambda i, e: (i, 0)),
            pl.BlockSpec((1, D, H), lambda i, e: (e, 0, 0)),
            pl.BlockSpec((1, 1, H), lambda i, e: (e, 0, 0)),
            pl.BlockSpec((1, H, H), lambda i, e: (e, 0, 0)),
            pl.BlockSpec((1, 1, H), lambda i, e: (e, 0, 0)),
            pl.BlockSpec((1, H, 2 * LANES), lambda i, e: (e, 0, 0)),
            pl.BlockSpec((1, 1, 2 * LANES), lambda i, e: (e, 0, 0)),
        ],
        out_specs=[
            pl.BlockSpec((TIL, LANES), lambda i, e: (i, 0)),
            pl.BlockSpec((TIL, 1), lambda i, e: (i, 0)),
        ],
        out_shape=[
            jax.ShapeDtypeStruct((N, LANES), jnp.float32),
            jax.ShapeDtypeStruct((N, 1), jnp.float32),
        ],
    )(x, gates, fc1_w, fc1_b[:, None, :], fc2_w, fc2_b[:, None, :], wpc, bpc)

    return p_hat, alpha.reshape(N), aux_vec[0, 0]


def kernel(x, gate_w, gate_b, fc1_w, fc1_b, fc2_w, fc2_b, fcp_w, fcp_b, fca_w, fca_b):
    return _run(x, gate_w, gate_b, fc1_w, fc1_b, fc2_w, fc2_b,
                fcp_w, fcp_b, fca_w, fca_b)
